# Initial kernel scaffold; baseline (speedup 1.0000x reference)
import jax
import jax.numpy as jnp
from jax.experimental import pallas as pl


def kernel(inputs, table):
    raise NotImplementedError("write your pallas kernel here")


# speedup vs baseline: 1.4771x; 1.4771x over previous
import functools

import jax
import jax.numpy as jnp
from jax import lax
from jax.experimental import pallas as pl
from jax.experimental.pallas import tpu as pltpu
from jax.experimental.pallas import tpu_sc as plsc


def _make_gather(V, D, B):
    info = plsc.get_sparse_core_info()
    NC, NS = info.num_cores, info.num_subcores
    NW = NC * NS
    assert B % (8 * NW) == 0
    b_per_w = B // NW
    CHUNK = 32
    NCHUNK = b_per_w // CHUNK
    NBUF = 2
    mesh = plsc.VectorSubcoreMesh(core_axis_name="c", subcore_axis_name="s")

    @functools.partial(
        pl.kernel,
        mesh=mesh,
        out_type=jax.ShapeDtypeStruct((B, D), jnp.float32),
        scratch_types=[
            pltpu.VMEM((b_per_w,), jnp.int32),
            pltpu.VMEM((NBUF, CHUNK, D), jnp.float32),
            pltpu.SemaphoreType.DMA((NBUF,)),
            pltpu.SemaphoreType.DMA((NBUF,)),
        ],
    )
    def k(table_hbm, idx_hbm, out_hbm, idx_v, rows_v, gsem, osem):
        wid = lax.axis_index("s") * NC + lax.axis_index("c")
        base = wid * b_per_w
        pltpu.sync_copy(idx_hbm.at[pl.ds(base, b_per_w)], idx_v)

        def gather(c):
            buf = c % NBUF
            return pltpu.async_copy(
                table_hbm.at[idx_v.at[pl.ds(c * CHUNK, CHUNK)]],
                rows_v.at[buf],
                gsem.at[buf],
            )

        def put(c):
            buf = c % NBUF
            return pltpu.async_copy(
                rows_v.at[buf],
                out_hbm.at[pl.ds(base + c * CHUNK, CHUNK)],
                osem.at[buf],
            )

        gathers = [None] * NCHUNK
        puts = [None] * NCHUNK
        gathers[0] = gather(0)
        for c in range(NCHUNK):
            gathers[c].wait()
            puts[c] = put(c)
            if c + 1 < NCHUNK:
                if c - 1 >= 0:
                    puts[c - 1].wait()
                gathers[c + 1] = gather(c + 1)
        puts[NCHUNK - 1].wait()

    return k


def kernel(inputs, table):
    Bt, S = inputs.shape
    V, D = table.shape
    flat_idx = inputs.reshape(-1).astype(jnp.int32)
    out = _make_gather(V, D, Bt * S)(table, flat_idx)
    return out.reshape(Bt, S, D)


# speedup vs baseline: 1.5257x; 1.0329x over previous
import functools

import jax
import jax.numpy as jnp
from jax import lax
from jax.experimental import pallas as pl
from jax.experimental.pallas import tpu as pltpu
from jax.experimental.pallas import tpu_sc as plsc


def _make_gather(V, D, B):
    info = plsc.get_sparse_core_info()
    NC, NS = info.num_cores, info.num_subcores
    NW = NC * NS
    assert B % (8 * NW) == 0
    b_per_w = B // NW
    CHUNK = 32
    NCHUNK = b_per_w // CHUNK
    NBUF = 3
    mesh = plsc.VectorSubcoreMesh(core_axis_name="c", subcore_axis_name="s")

    @functools.partial(
        pl.kernel,
        mesh=mesh,
        out_type=jax.ShapeDtypeStruct((B, D), jnp.float32),
        scratch_types=[
            pltpu.VMEM((b_per_w,), jnp.int32),
            pltpu.VMEM((NBUF, CHUNK, D), jnp.float32),
            pltpu.SemaphoreType.DMA((NBUF,)),
            pltpu.SemaphoreType.DMA((NBUF,)),
        ],
    )
    def k(table_hbm, idx_hbm, out_hbm, idx_v, rows_v, gsem, osem):
        wid = lax.axis_index("s") * NC + lax.axis_index("c")
        base = wid * b_per_w
        pltpu.sync_copy(idx_hbm.at[pl.ds(base, b_per_w)], idx_v)

        def gather(c):
            buf = c % NBUF
            return pltpu.async_copy(
                table_hbm.at[idx_v.at[pl.ds(c * CHUNK, CHUNK)]],
                rows_v.at[buf],
                gsem.at[buf],
            )

        def put(c):
            buf = c % NBUF
            return pltpu.async_copy(
                rows_v.at[buf],
                out_hbm.at[pl.ds(base + c * CHUNK, CHUNK)],
                osem.at[buf],
            )

        gathers = [None] * NCHUNK
        puts = [None] * NCHUNK
        put_done = [False] * NCHUNK
        gathers[0] = gather(0)
        if NCHUNK > 1:
            gathers[1] = gather(1)
        for c in range(NCHUNK):
            gathers[c].wait()
            puts[c] = put(c)
            if c + 2 < NCHUNK:
                if c - 1 >= 0:
                    puts[c - 1].wait()
                    put_done[c - 1] = True
                gathers[c + 2] = gather(c + 2)
        for c in range(NCHUNK):
            if not put_done[c]:
                puts[c].wait()

    return k


def kernel(inputs, table):
    Bt, S = inputs.shape
    V, D = table.shape
    flat_idx = inputs.reshape(-1).astype(jnp.int32)
    out = _make_gather(V, D, Bt * S)(table, flat_idx)
    return out.reshape(Bt, S, D)


# speedup vs baseline: 1.5981x; 1.0475x over previous
import functools

import jax
import jax.numpy as jnp
from jax import lax
from jax.experimental import pallas as pl
from jax.experimental.pallas import tpu as pltpu
from jax.experimental.pallas import tpu_sc as plsc


def _make_gather(V, D, B):
    info = plsc.get_sparse_core_info()
    NC, NS = info.num_cores, info.num_subcores
    NW = NC * NS
    assert B % (8 * NW) == 0
    b_per_w = B // NW
    CHUNK = 16
    NCHUNK = b_per_w // CHUNK
    NBUF = 6
    mesh = plsc.VectorSubcoreMesh(core_axis_name="c", subcore_axis_name="s")

    @functools.partial(
        pl.kernel,
        mesh=mesh,
        out_type=jax.ShapeDtypeStruct((B, D), jnp.float32),
        scratch_types=[
            pltpu.VMEM((b_per_w,), jnp.int32),
            pltpu.VMEM((NBUF, CHUNK, D), jnp.float32),
            pltpu.SemaphoreType.DMA((NBUF,)),
            pltpu.SemaphoreType.DMA((NBUF,)),
        ],
    )
    def k(table_hbm, idx_hbm, out_hbm, idx_v, rows_v, gsem, osem):
        wid = lax.axis_index("s") * NC + lax.axis_index("c")
        base = wid * b_per_w
        pltpu.sync_copy(idx_hbm.at[pl.ds(base, b_per_w)], idx_v)

        def gather(c):
            buf = c % NBUF
            return pltpu.async_copy(
                table_hbm.at[idx_v.at[pl.ds(c * CHUNK, CHUNK)]],
                rows_v.at[buf],
                gsem.at[buf],
            )

        def put(c):
            buf = c % NBUF
            return pltpu.async_copy(
                rows_v.at[buf],
                out_hbm.at[pl.ds(base + c * CHUNK, CHUNK)],
                osem.at[buf],
            )

        DEPTH = NBUF - 1
        gathers = [None] * NCHUNK
        puts = [None] * NCHUNK
        put_done = [False] * NCHUNK
        for c in range(min(DEPTH, NCHUNK)):
            gathers[c] = gather(c)
        for c in range(NCHUNK):
            gathers[c].wait()
            puts[c] = put(c)
            if c + DEPTH < NCHUNK:
                if c - 1 >= 0:
                    puts[c - 1].wait()
                    put_done[c - 1] = True
                gathers[c + DEPTH] = gather(c + DEPTH)
        for c in range(NCHUNK):
            if not put_done[c]:
                puts[c].wait()

    return k


def kernel(inputs, table):
    Bt, S = inputs.shape
    V, D = table.shape
    flat_idx = inputs.reshape(-1).astype(jnp.int32)
    out = _make_gather(V, D, Bt * S)(table, flat_idx)
    return out.reshape(Bt, S, D)
